# baseline (speedup 1.0000x reference)
import jax
import jax.numpy as jnp
from jax import lax
from jax.experimental import pallas as pl
from jax.experimental.pallas import tpu as pltpu
from jax.experimental.pallas import tpu_sc as plsc

B = 512
G = 978
N = B * G
E = N * 16
N_TILES = 16
E_PER_TILE = E // N_TILES
CHUNK = 3072
N_CHUNKS = E_PER_TILE // CHUNK
N_PAIRS = (N_CHUNKS + 1) // 2
N_PER_TILE = N // N_TILES
STAGE = N_PER_TILE // 2


def _sc_segment_sum(x1, x2, edges_hbm_arr, zeros_n):
    mesh = plsc.VectorSubcoreMesh(core_axis_name="c", subcore_axis_name="s")

    def body(x1_hbm, x2_hbm, edges_hbm, zeros_hbm, out_hbm,
             xsh, aggsh, s0, s1, d0, d1, v0, v1, stg,
             sem_s0, sem_s1, sem_d0, sem_d1, sem_g0, sem_g1,
             sem_c0, sem_c1):
        c = lax.axis_index("c")
        s = lax.axis_index("s")
        n0 = s * N_PER_TILE

        for k in range(N_PER_TILE // STAGE):
            p0 = n0 + k * STAGE

            @pl.when(c == 0)
            def _():
                pltpu.sync_copy(x1_hbm.at[pl.ds(p0, STAGE)], stg)

            @pl.when(c != 0)
            def _():
                pltpu.sync_copy(x2_hbm.at[pl.ds(p0, STAGE)], stg)

            pltpu.sync_copy(stg, xsh.at[pl.ds(p0, STAGE)])
            pltpu.sync_copy(zeros_hbm.at[pl.ds(p0, STAGE)], stg)
            pltpu.sync_copy(stg, aggsh.at[pl.ds(p0, STAGE)])
        plsc.subcore_barrier()

        tile_e0 = s * E_PER_TILE

        def dma_in(i, buf, sem, row):
            e0 = tile_e0 + i * CHUNK
            pltpu.async_copy(edges_hbm.at[row, pl.ds(e0, CHUNK)], buf, sem)

        def dma_wait(i, buf, sem, row):
            e0 = tile_e0 + i * CHUNK
            pltpu.make_async_copy(edges_hbm.at[row, pl.ds(e0, CHUNK)], buf,
                                  sem).wait()

        dma_in(0, s0, sem_s0, 0)
        dma_in(0, d0, sem_d0, 1)
        dma_in(1, s1, sem_s1, 0)
        dma_in(1, d1, sem_d1, 1)

        def step(k, carry):
            a = 2 * k
            b = a + 1
            dma_wait(a, s0, sem_s0, 0)
            dma_wait(a, d0, sem_d0, 1)
            g_a = pltpu.async_copy(xsh.at[s0], v0, sem_g0)

            @pl.when(b < N_CHUNKS)
            def _():
                dma_wait(b, s1, sem_s1, 0)
                dma_wait(b, d1, sem_d1, 1)
                pltpu.async_copy(xsh.at[s1], v1, sem_g1)

            g_a.wait()
            c_a = pltpu.async_copy(v0, aggsh.at[d0], sem_c0, add=True)

            @pl.when(a + 2 < N_CHUNKS)
            def _():
                dma_in(a + 2, s0, sem_s0, 0)

            @pl.when(b < N_CHUNKS)
            def _():
                pltpu.make_async_copy(xsh.at[s1], v1, sem_g1).wait()
                pltpu.async_copy(v1, aggsh.at[d1], sem_c1, add=True)

            @pl.when(b + 2 < N_CHUNKS)
            def _():
                dma_in(b + 2, s1, sem_s1, 0)

            c_a.wait()

            @pl.when(a + 2 < N_CHUNKS)
            def _():
                dma_in(a + 2, d0, sem_d0, 1)

            @pl.when(b < N_CHUNKS)
            def _():
                pltpu.make_async_copy(v1, aggsh.at[d1], sem_c1).wait()

            @pl.when(b + 2 < N_CHUNKS)
            def _():
                dma_in(b + 2, d1, sem_d1, 1)

            return carry

        lax.fori_loop(0, N_PAIRS, step, 0)
        plsc.subcore_barrier()
        for k in range(N_PER_TILE // STAGE):
            p0 = n0 + k * STAGE
            pltpu.sync_copy(aggsh.at[pl.ds(p0, STAGE)], stg)
            pltpu.sync_copy(stg, out_hbm.at[pl.ds(c * N + p0, STAGE)])

    f = pl.kernel(
        body,
        out_type=jax.ShapeDtypeStruct((2 * N,), jnp.float32),
        mesh=mesh,
        scratch_types=[
            pltpu.VMEM_SHARED((N,), jnp.float32),
            pltpu.VMEM_SHARED((N,), jnp.float32),
            pltpu.VMEM((CHUNK,), jnp.int32),
            pltpu.VMEM((CHUNK,), jnp.int32),
            pltpu.VMEM((CHUNK,), jnp.int32),
            pltpu.VMEM((CHUNK,), jnp.int32),
            pltpu.VMEM((CHUNK,), jnp.float32),
            pltpu.VMEM((CHUNK,), jnp.float32),
            pltpu.VMEM((STAGE,), jnp.float32),
        ] + [pltpu.SemaphoreType.DMA] * 8,
    )
    return f(x1, x2, edges_hbm_arr, zeros_n)


def _tc_head(agg, others, gcn_w, gcn_b, fc1_w, fc1_b, fc2_w, fc2_b,
             mlp1_w, mlp1_b, mlp2_w, mlp2_b, interpret=False):
    def body(agg_ref, oth_ref, gw_ref, gb_ref, w1_ref, b1_ref, w2_ref, b2_ref,
             m1w_ref, m1b_ref, m2w_ref, m2b_ref, out_ref):
        gw = gw_ref[...]
        gb = gb_ref[...]

        def enc(a):
            h = jnp.maximum(a * gw + gb, 0.0)
            h = lax.dot_general(h, w1_ref[...], (((1,), (1,)), ((), ())),
                                preferred_element_type=jnp.float32)
            h = jnp.maximum(h + b1_ref[...], 0.0)
            o = lax.dot_general(h, w2_ref[...], (((1,), (1,)), ((), ())),
                                preferred_element_type=jnp.float32)
            return o + b2_ref[...]

        o1 = enc(agg_ref[0])
        o2 = enc(agg_ref[1])
        p1 = o1 - jnp.mean(o1, axis=1, keepdims=True)
        p2 = o2 - jnp.mean(o2, axis=1, keepdims=True)
        n1 = jnp.sum(p1 * p1, axis=1, keepdims=True)
        n2 = jnp.sum(p2 * p2, axis=1, keepdims=True)
        p12 = jnp.sum(p1 * p2, axis=1, keepdims=True)
        r = p12 / jnp.sqrt(n1 * n2)
        r2 = r * r
        cat = jnp.concatenate([r2, oth_ref[...]], axis=1)
        z = lax.dot_general(cat, m1w_ref[...], (((1,), (1,)), ((), ())),
                            preferred_element_type=jnp.float32)
        z = jnp.maximum(z + m1b_ref[...], 0.0)
        out = lax.dot_general(z, m2w_ref[...], (((1,), (1,)), ((), ())),
                              preferred_element_type=jnp.float32)
        out_ref[...] = out + m2b_ref[...]

    return pl.pallas_call(
        body,
        out_shape=jax.ShapeDtypeStruct((B, 2), jnp.float32),
        interpret=interpret,
    )(agg, others, gcn_w, gcn_b, fc1_w, fc1_b, fc2_w, fc2_b,
      mlp1_w, mlp1_b, mlp2_w, mlp2_b)


def kernel(input1, input2, edges, input_others, gcn_w, gcn_b,
           fc1_w, fc1_b, fc2_w, fc2_b, mlp1_w, mlp1_b, mlp2_w, mlp2_b):
    x1 = input1.reshape(-1)
    x2 = input2.reshape(-1)
    zeros_n = jnp.zeros((N,), jnp.float32)
    agg = _sc_segment_sum(x1, x2, edges, zeros_n)
    return _tc_head(agg.reshape(2, B, G), input_others,
                    gcn_w, gcn_b.reshape(1, 1),
                    fc1_w, fc1_b.reshape(1, -1),
                    fc2_w, fc2_b.reshape(1, -1),
                    mlp1_w, mlp1_b.reshape(1, -1),
                    mlp2_w, mlp2_b.reshape(1, -1))


# speedup vs baseline: 1.3249x; 1.3249x over previous
import jax
import jax.numpy as jnp
from jax import lax
from jax.experimental import pallas as pl
from jax.experimental.pallas import tpu as pltpu
from jax.experimental.pallas import tpu_sc as plsc

B = 512
G = 978
N = B * G
E = N * 16
N_TILES = 16
E_PER_TILE = E // N_TILES
CHUNK = 11392
N_CHUNKS = 44
TAIL = E_PER_TILE - 43 * CHUNK
N_PAIRS = N_CHUNKS // 2
N_PER_TILE = N // N_TILES
STAGE = N_PER_TILE // 4


def _sc_segment_sum(x1, x2, edges_hbm_arr, zeros_n):
    mesh = plsc.VectorSubcoreMesh(core_axis_name="c", subcore_axis_name="s")

    def body(x1_hbm, x2_hbm, edges_hbm, zeros_hbm, out_hbm,
             xsh, aggsh, s0, s1, d0, d1, v0, v1,
             sem_s0, sem_s1, sem_d0, sem_d1, sem_g0, sem_g1,
             sem_c0, sem_c1):
        c = lax.axis_index("c")
        s = lax.axis_index("s")
        n0 = s * N_PER_TILE

        stg = v0.at[pl.ds(0, STAGE)]
        for k in range(N_PER_TILE // STAGE):
            p0 = n0 + k * STAGE

            @pl.when(c == 0)
            def _():
                pltpu.sync_copy(x1_hbm.at[pl.ds(p0, STAGE)], stg)

            @pl.when(c != 0)
            def _():
                pltpu.sync_copy(x2_hbm.at[pl.ds(p0, STAGE)], stg)

            pltpu.sync_copy(stg, xsh.at[pl.ds(p0, STAGE)])
            pltpu.sync_copy(zeros_hbm.at[pl.ds(p0, STAGE)], stg)
            pltpu.sync_copy(stg, aggsh.at[pl.ds(p0, STAGE)])
        plsc.subcore_barrier()

        tile_e0 = s * E_PER_TILE
        zvec = jnp.zeros((16,), jnp.float32)

        def dma_in(i, buf, sem, row, tail):
            e0 = tile_e0 + i * CHUNK
            n = TAIL if tail else CHUNK
            pltpu.async_copy(edges_hbm.at[row, pl.ds(e0, n)],
                             buf.at[pl.ds(0, n)] if tail else buf, sem)

        def dma_wait(i, buf, sem, row, tail):
            e0 = tile_e0 + i * CHUNK
            n = TAIL if tail else CHUNK
            pltpu.make_async_copy(edges_hbm.at[row, pl.ds(e0, n)],
                                  buf.at[pl.ds(0, n)] if tail else buf,
                                  sem).wait()

        dma_in(0, s0, sem_s0, 0, False)
        dma_in(0, d0, sem_d0, 1, False)
        dma_in(1, s1, sem_s1, 0, False)
        dma_in(1, d1, sem_d1, 1, False)

        def step(k, carry):
            a = 2 * k
            b = a + 1
            is_tail = b == N_CHUNKS - 1
            dma_wait(a, s0, sem_s0, 0, False)
            dma_wait(a, d0, sem_d0, 1, False)
            g_a = pltpu.async_copy(xsh.at[s0], v0, sem_g0)

            @pl.when(jnp.logical_not(is_tail))
            def _():
                dma_wait(b, s1, sem_s1, 0, False)
                dma_wait(b, d1, sem_d1, 1, False)

            @pl.when(is_tail)
            def _():
                dma_wait(b, s1, sem_s1, 0, True)
                dma_wait(b, d1, sem_d1, 1, True)

            g_b = pltpu.async_copy(xsh.at[s1], v1, sem_g1)

            g_a.wait()
            c_a = pltpu.async_copy(v0, aggsh.at[d0], sem_c0, add=True)

            @pl.when(a + 2 < N_CHUNKS)
            def _():
                dma_in(a + 2, s0, sem_s0, 0, False)

            g_b.wait()

            @pl.when(is_tail)
            def _():
                for j in range(TAIL, CHUNK, 16):
                    v1[pl.ds(j, 16)] = zvec

            c_b = pltpu.async_copy(v1, aggsh.at[d1], sem_c1, add=True)

            @pl.when(b + 2 < N_CHUNKS - 1)
            def _():
                dma_in(b + 2, s1, sem_s1, 0, False)

            @pl.when(b + 2 == N_CHUNKS - 1)
            def _():
                dma_in(b + 2, s1, sem_s1, 0, True)

            c_a.wait()

            @pl.when(a + 2 < N_CHUNKS)
            def _():
                dma_in(a + 2, d0, sem_d0, 1, False)

            c_b.wait()

            @pl.when(b + 2 < N_CHUNKS - 1)
            def _():
                dma_in(b + 2, d1, sem_d1, 1, False)

            @pl.when(b + 2 == N_CHUNKS - 1)
            def _():
                dma_in(b + 2, d1, sem_d1, 1, True)

            return carry

        lax.fori_loop(0, N_PAIRS, step, 0)
        plsc.subcore_barrier()
        for k in range(N_PER_TILE // STAGE):
            p0 = n0 + k * STAGE
            pltpu.sync_copy(aggsh.at[pl.ds(p0, STAGE)], stg)
            pltpu.sync_copy(stg, out_hbm.at[pl.ds(c * N + p0, STAGE)])

    f = pl.kernel(
        body,
        out_type=jax.ShapeDtypeStruct((2 * N,), jnp.float32),
        mesh=mesh,
        scratch_types=[
            pltpu.VMEM_SHARED((N,), jnp.float32),
            pltpu.VMEM_SHARED((N,), jnp.float32),
            pltpu.VMEM((CHUNK,), jnp.int32),
            pltpu.VMEM((CHUNK,), jnp.int32),
            pltpu.VMEM((CHUNK,), jnp.int32),
            pltpu.VMEM((CHUNK,), jnp.int32),
            pltpu.VMEM((CHUNK,), jnp.float32),
            pltpu.VMEM((CHUNK,), jnp.float32),
        ] + [pltpu.SemaphoreType.DMA] * 8,
    )
    return f(x1, x2, edges_hbm_arr, zeros_n)


def _tc_head(agg, others, gcn_w, gcn_b, fc1_w, fc1_b, fc2_w, fc2_b,
             mlp1_w, mlp1_b, mlp2_w, mlp2_b, interpret=False):
    def body(agg_ref, oth_ref, gw_ref, gb_ref, w1_ref, b1_ref, w2_ref, b2_ref,
             m1w_ref, m1b_ref, m2w_ref, m2b_ref, out_ref):
        gw = gw_ref[...]
        gb = gb_ref[...]

        def enc(a):
            h = jnp.maximum(a * gw + gb, 0.0)
            h = lax.dot_general(h, w1_ref[...], (((1,), (1,)), ((), ())),
                                preferred_element_type=jnp.float32)
            h = jnp.maximum(h + b1_ref[...], 0.0)
            o = lax.dot_general(h, w2_ref[...], (((1,), (1,)), ((), ())),
                                preferred_element_type=jnp.float32)
            return o + b2_ref[...]

        o1 = enc(agg_ref[0])
        o2 = enc(agg_ref[1])
        p1 = o1 - jnp.mean(o1, axis=1, keepdims=True)
        p2 = o2 - jnp.mean(o2, axis=1, keepdims=True)
        n1 = jnp.sum(p1 * p1, axis=1, keepdims=True)
        n2 = jnp.sum(p2 * p2, axis=1, keepdims=True)
        p12 = jnp.sum(p1 * p2, axis=1, keepdims=True)
        r = p12 / jnp.sqrt(n1 * n2)
        r2 = r * r
        cat = jnp.concatenate([r2, oth_ref[...]], axis=1)
        z = lax.dot_general(cat, m1w_ref[...], (((1,), (1,)), ((), ())),
                            preferred_element_type=jnp.float32)
        z = jnp.maximum(z + m1b_ref[...], 0.0)
        out = lax.dot_general(z, m2w_ref[...], (((1,), (1,)), ((), ())),
                              preferred_element_type=jnp.float32)
        out_ref[...] = out + m2b_ref[...]

    return pl.pallas_call(
        body,
        out_shape=jax.ShapeDtypeStruct((B, 2), jnp.float32),
        interpret=interpret,
    )(agg, others, gcn_w, gcn_b, fc1_w, fc1_b, fc2_w, fc2_b,
      mlp1_w, mlp1_b, mlp2_w, mlp2_b)


def kernel(input1, input2, edges, input_others, gcn_w, gcn_b,
           fc1_w, fc1_b, fc2_w, fc2_b, mlp1_w, mlp1_b, mlp2_w, mlp2_b):
    x1 = input1.reshape(-1)
    x2 = input2.reshape(-1)
    zeros_n = jnp.zeros((N,), jnp.float32)
    agg = _sc_segment_sum(x1, x2, edges, zeros_n)
    return _tc_head(agg.reshape(2, B, G), input_others,
                    gcn_w, gcn_b.reshape(1, 1),
                    fc1_w, fc1_b.reshape(1, -1),
                    fc2_w, fc2_b.reshape(1, -1),
                    mlp1_w, mlp1_b.reshape(1, -1),
                    mlp2_w, mlp2_b.reshape(1, -1))


# speedup vs baseline: 1.5513x; 1.1709x over previous
import jax
import jax.numpy as jnp
from jax import lax
from jax.experimental import pallas as pl
from jax.experimental.pallas import tpu as pltpu
from jax.experimental.pallas import tpu_sc as plsc

B = 512
G = 978
N = B * G
E = N * 16
N_TILES = 16
E_PER_TILE = E // N_TILES
CHUNK = 20864
N_CHUNKS = E_PER_TILE // CHUNK
N_PER_TILE = N // N_TILES
STAGE = N_PER_TILE // 2


def _sc_segment_sum(x1, x2, edges_r, zeros_n):
    mesh = plsc.VectorSubcoreMesh(core_axis_name="c", subcore_axis_name="s")

    def body(x1_hbm, x2_hbm, edges_hbm, zeros_hbm, out_hbm,
             xsh, aggsh, srcb, dstb, valb, sem_s, sem_d):
        c = lax.axis_index("c")
        s = lax.axis_index("s")
        n0 = s * N_PER_TILE

        stage = valb.at[pl.ds(0, STAGE)]
        for k in range(N_PER_TILE // STAGE):
            p0 = n0 + k * STAGE

            @pl.when(c == 0)
            def _():
                pltpu.sync_copy(x1_hbm.at[pl.ds(p0, STAGE)], stage)

            @pl.when(c != 0)
            def _():
                pltpu.sync_copy(x2_hbm.at[pl.ds(p0, STAGE)], stage)

            pltpu.sync_copy(stage, xsh.at[pl.ds(p0, STAGE)])
            pltpu.sync_copy(zeros_hbm.at[pl.ds(p0, STAGE)], stage)
            pltpu.sync_copy(stage, aggsh.at[pl.ds(p0, STAGE)])
        plsc.subcore_barrier()

        tile_e0 = s * E_PER_TILE

        def chunk(i, carry):
            e0 = tile_e0 + i * CHUNK
            cp_s = pltpu.async_copy(edges_hbm.at[0, pl.ds(e0, CHUNK)],
                                    srcb, sem_s)
            cp_d = pltpu.async_copy(edges_hbm.at[1, pl.ds(e0, CHUNK)],
                                    dstb, sem_d)
            cp_s.wait()
            cp_d.wait()
            pltpu.sync_copy(xsh.at[srcb], valb)
            pltpu.sync_copy(valb, aggsh.at[dstb], add=True)
            return carry

        lax.fori_loop(0, N_CHUNKS, chunk, 0)
        plsc.subcore_barrier()
        for k in range(N_PER_TILE // STAGE):
            p0 = n0 + k * STAGE
            pltpu.sync_copy(aggsh.at[pl.ds(p0, STAGE)], stage)
            pltpu.sync_copy(stage, out_hbm.at[pl.ds(c * N + p0, STAGE)])

    f = pl.kernel(
        body,
        out_type=jax.ShapeDtypeStruct((2 * N,), jnp.float32),
        mesh=mesh,
        scratch_types=[
            pltpu.VMEM_SHARED((N,), jnp.float32),
            pltpu.VMEM_SHARED((N,), jnp.float32),
            pltpu.VMEM((CHUNK,), jnp.int32),
            pltpu.VMEM((CHUNK,), jnp.int32),
            pltpu.VMEM((CHUNK,), jnp.float32),
            pltpu.SemaphoreType.DMA,
            pltpu.SemaphoreType.DMA,
        ],
    )
    return f(x1, x2, edges_r, zeros_n)


def _tc_head(agg, others, gcn_w, gcn_b, fc1_w, fc1_b, fc2_w, fc2_b,
             mlp1_w, mlp1_b, mlp2_w, mlp2_b, interpret=False):
    def body(agg_ref, oth_ref, gw_ref, gb_ref, w1_ref, b1_ref, w2_ref, b2_ref,
             m1w_ref, m1b_ref, m2w_ref, m2b_ref, out_ref):
        gw = gw_ref[...]
        gb = gb_ref[...]

        def enc(a):
            h = jnp.maximum(a * gw + gb, 0.0)
            h = lax.dot_general(h, w1_ref[...], (((1,), (1,)), ((), ())),
                                preferred_element_type=jnp.float32)
            h = jnp.maximum(h + b1_ref[...], 0.0)
            o = lax.dot_general(h, w2_ref[...], (((1,), (1,)), ((), ())),
                                preferred_element_type=jnp.float32)
            return o + b2_ref[...]

        o1 = enc(agg_ref[0])
        o2 = enc(agg_ref[1])
        p1 = o1 - jnp.mean(o1, axis=1, keepdims=True)
        p2 = o2 - jnp.mean(o2, axis=1, keepdims=True)
        n1 = jnp.sum(p1 * p1, axis=1, keepdims=True)
        n2 = jnp.sum(p2 * p2, axis=1, keepdims=True)
        p12 = jnp.sum(p1 * p2, axis=1, keepdims=True)
        r = p12 / jnp.sqrt(n1 * n2)
        r2 = r * r
        cat = jnp.concatenate([r2, oth_ref[...]], axis=1)
        z = lax.dot_general(cat, m1w_ref[...], (((1,), (1,)), ((), ())),
                            preferred_element_type=jnp.float32)
        z = jnp.maximum(z + m1b_ref[...], 0.0)
        out = lax.dot_general(z, m2w_ref[...], (((1,), (1,)), ((), ())),
                              preferred_element_type=jnp.float32)
        out_ref[...] = out + m2b_ref[...]

    return pl.pallas_call(
        body,
        out_shape=jax.ShapeDtypeStruct((B, 2), jnp.float32),
        interpret=interpret,
    )(agg, others, gcn_w, gcn_b, fc1_w, fc1_b, fc2_w, fc2_b,
      mlp1_w, mlp1_b, mlp2_w, mlp2_b)


def kernel(input1, input2, edges, input_others, gcn_w, gcn_b,
           fc1_w, fc1_b, fc2_w, fc2_b, mlp1_w, mlp1_b, mlp2_w, mlp2_b):
    x1 = input1.reshape(-1)
    x2 = input2.reshape(-1)
    zeros_n = jnp.zeros((N,), jnp.float32)
    agg = _sc_segment_sum(x1, x2, edges, zeros_n)
    return _tc_head(agg.reshape(2, B, G), input_others,
                    gcn_w, gcn_b.reshape(1, 1),
                    fc1_w, fc1_b.reshape(1, -1),
                    fc2_w, fc2_b.reshape(1, -1),
                    mlp1_w, mlp1_b.reshape(1, -1),
                    mlp2_w, mlp2_b.reshape(1, -1))


# speedup vs baseline: 1.7403x; 1.1218x over previous
import jax
import jax.numpy as jnp
from jax import lax
from jax.experimental import pallas as pl
from jax.experimental.pallas import tpu as pltpu
from jax.experimental.pallas import tpu_sc as plsc

B = 512
G = 978
N = B * G
E = N * 16
N_TILES = 16
E_PER_TILE = E // N_TILES
CHUNK = 20864
N_CHUNKS = E_PER_TILE // CHUNK
N_PER_TILE = N // N_TILES
STAGE = N_PER_TILE // 2


def _sc_segment_sum(x1, x2, edges_r):
    mesh = plsc.VectorSubcoreMesh(core_axis_name="c", subcore_axis_name="s")

    def body(x1_hbm, x2_hbm, edges_hbm, out_hbm,
             xsh, aggsh, srcb, dstb, valb, sem_s, sem_d):
        c = lax.axis_index("c")
        s = lax.axis_index("s")
        n0 = s * N_PER_TILE

        stage = valb.at[pl.ds(0, STAGE)]
        zvec = jnp.zeros((16,), jnp.float32)

        def zfill(j, carry):
            valb[pl.ds(j * 16, 16)] = zvec
            return carry

        lax.fori_loop(0, STAGE // 16, zfill, 0)
        for k in range(N_PER_TILE // STAGE):
            p0 = n0 + k * STAGE
            pltpu.sync_copy(stage, aggsh.at[pl.ds(p0, STAGE)])
        for k in range(N_PER_TILE // STAGE):
            p0 = n0 + k * STAGE

            @pl.when(c == 0)
            def _():
                pltpu.sync_copy(x1_hbm.at[pl.ds(p0, STAGE)], stage)

            @pl.when(c != 0)
            def _():
                pltpu.sync_copy(x2_hbm.at[pl.ds(p0, STAGE)], stage)

            pltpu.sync_copy(stage, xsh.at[pl.ds(p0, STAGE)])
        plsc.subcore_barrier()

        tile_e0 = s * E_PER_TILE

        pltpu.async_copy(edges_hbm.at[0, pl.ds(tile_e0, CHUNK)], srcb, sem_s)

        def chunk(i, carry):
            e0 = tile_e0 + i * CHUNK
            pltpu.async_copy(edges_hbm.at[1, pl.ds(e0, CHUNK)], dstb, sem_d)
            pltpu.make_async_copy(edges_hbm.at[0, pl.ds(e0, CHUNK)], srcb,
                                  sem_s).wait()
            pltpu.sync_copy(xsh.at[srcb], valb)

            @pl.when(i < N_CHUNKS - 1)
            def _():
                pltpu.async_copy(edges_hbm.at[0, pl.ds(e0 + CHUNK, CHUNK)],
                                 srcb, sem_s)

            pltpu.make_async_copy(edges_hbm.at[1, pl.ds(e0, CHUNK)], dstb,
                                  sem_d).wait()
            pltpu.sync_copy(valb, aggsh.at[dstb], add=True)
            return carry

        lax.fori_loop(0, N_CHUNKS, chunk, 0)
        plsc.subcore_barrier()
        for k in range(N_PER_TILE // STAGE):
            p0 = n0 + k * STAGE
            pltpu.sync_copy(aggsh.at[pl.ds(p0, STAGE)], stage)
            pltpu.sync_copy(stage, out_hbm.at[pl.ds(c * N + p0, STAGE)])

    f = pl.kernel(
        body,
        out_type=jax.ShapeDtypeStruct((2 * N,), jnp.float32),
        mesh=mesh,
        scratch_types=[
            pltpu.VMEM_SHARED((N,), jnp.float32),
            pltpu.VMEM_SHARED((N,), jnp.float32),
            pltpu.VMEM((CHUNK,), jnp.int32),
            pltpu.VMEM((CHUNK,), jnp.int32),
            pltpu.VMEM((CHUNK,), jnp.float32),
            pltpu.SemaphoreType.DMA,
            pltpu.SemaphoreType.DMA,
        ],
    )
    return f(x1, x2, edges_r)


def _tc_head(agg, others, gcn_w, gcn_b, fc1_w, fc1_b, fc2_w, fc2_b,
             mlp1_w, mlp1_b, mlp2_w, mlp2_b, interpret=False):
    def body(agg_ref, oth_ref, gw_ref, gb_ref, w1_ref, b1_ref, w2_ref, b2_ref,
             m1w_ref, m1b_ref, m2w_ref, m2b_ref, out_ref):
        gw = gw_ref[...]
        gb = gb_ref[...]

        def enc(a):
            h = jnp.maximum(a * gw + gb, 0.0)
            h = lax.dot_general(h, w1_ref[...], (((1,), (1,)), ((), ())),
                                preferred_element_type=jnp.float32)
            h = jnp.maximum(h + b1_ref[...], 0.0)
            o = lax.dot_general(h, w2_ref[...], (((1,), (1,)), ((), ())),
                                preferred_element_type=jnp.float32)
            return o + b2_ref[...]

        o1 = enc(agg_ref[0])
        o2 = enc(agg_ref[1])
        p1 = o1 - jnp.mean(o1, axis=1, keepdims=True)
        p2 = o2 - jnp.mean(o2, axis=1, keepdims=True)
        n1 = jnp.sum(p1 * p1, axis=1, keepdims=True)
        n2 = jnp.sum(p2 * p2, axis=1, keepdims=True)
        p12 = jnp.sum(p1 * p2, axis=1, keepdims=True)
        r = p12 / jnp.sqrt(n1 * n2)
        r2 = r * r
        cat = jnp.concatenate([r2, oth_ref[...]], axis=1)
        z = lax.dot_general(cat, m1w_ref[...], (((1,), (1,)), ((), ())),
                            preferred_element_type=jnp.float32)
        z = jnp.maximum(z + m1b_ref[...], 0.0)
        out = lax.dot_general(z, m2w_ref[...], (((1,), (1,)), ((), ())),
                              preferred_element_type=jnp.float32)
        out_ref[...] = out + m2b_ref[...]

    return pl.pallas_call(
        body,
        out_shape=jax.ShapeDtypeStruct((B, 2), jnp.float32),
        interpret=interpret,
    )(agg, others, gcn_w, gcn_b, fc1_w, fc1_b, fc2_w, fc2_b,
      mlp1_w, mlp1_b, mlp2_w, mlp2_b)


def kernel(input1, input2, edges, input_others, gcn_w, gcn_b,
           fc1_w, fc1_b, fc2_w, fc2_b, mlp1_w, mlp1_b, mlp2_w, mlp2_b):
    x1 = input1.reshape(-1)
    x2 = input2.reshape(-1)
    agg = _sc_segment_sum(x1, x2, edges)
    return _tc_head(agg.reshape(2, B, G), input_others,
                    gcn_w, gcn_b.reshape(1, 1),
                    fc1_w, fc1_b.reshape(1, -1),
                    fc2_w, fc2_b.reshape(1, -1),
                    mlp1_w, mlp1_b.reshape(1, -1),
                    mlp2_w, mlp2_b.reshape(1, -1))


# speedup vs baseline: 1.7414x; 1.0006x over previous
import jax
import jax.numpy as jnp
from jax import lax
from jax.experimental import pallas as pl
from jax.experimental.pallas import tpu as pltpu
from jax.experimental.pallas import tpu_sc as plsc

B = 512
G = 978
N = B * G
E = N * 16
N_TILES = 16
E_PER_TILE = E // N_TILES
CHUNK = 20864
N_CHUNKS = E_PER_TILE // CHUNK
N_PER_TILE = N // N_TILES
STAGE = N_PER_TILE // 2


def _sc_segment_sum(x1, x2, edges_r):
    mesh = plsc.VectorSubcoreMesh(core_axis_name="c", subcore_axis_name="s")

    def body(x1_hbm, x2_hbm, edges_hbm, out_hbm,
             xsh, aggsh, srcb, dstb, valb, sem_s, sem_d):
        c = lax.axis_index("c")
        s = lax.axis_index("s")
        n0 = s * N_PER_TILE

        HALF = STAGE // 2
        slots = [valb.at[pl.ds(0, HALF)], valb.at[pl.ds(HALF, HALF)]]
        sems = [sem_s, sem_d]
        zvec = jnp.zeros((16,), jnp.float32)

        def zfill(j, carry):
            valb[pl.ds(j * 16, 16)] = zvec
            return carry

        lax.fori_loop(0, HALF // 16, zfill, 0)
        for k in range(4):
            pltpu.async_copy(slots[0], aggsh.at[pl.ds(n0 + k * HALF, HALF)],
                             sem_s)
        for k in range(4):
            pltpu.make_async_copy(slots[0], aggsh.at[pl.ds(n0, HALF)],
                                  sem_s).wait()

        def hbm_to_spmem(dst_sh):
            for k in range(4):
                p0 = n0 + k * HALF
                slot, sem = slots[k % 2], sems[k % 2]
                if k >= 2:
                    pltpu.make_async_copy(
                        slot, dst_sh.at[pl.ds(n0 + (k - 2) * HALF, HALF)],
                        sem).wait()

                @pl.when(c == 0)
                def _():
                    pltpu.async_copy(x1_hbm.at[pl.ds(p0, HALF)], slot,
                                     sem).wait()

                @pl.when(c != 0)
                def _():
                    pltpu.async_copy(x2_hbm.at[pl.ds(p0, HALF)], slot,
                                     sem).wait()

                pltpu.async_copy(slot, dst_sh.at[pl.ds(p0, HALF)], sem)
            for k in range(2, 4):
                pltpu.make_async_copy(
                    slots[k % 2], dst_sh.at[pl.ds(n0 + k * HALF, HALF)],
                    sems[k % 2]).wait()

        hbm_to_spmem(xsh)
        plsc.subcore_barrier()

        tile_e0 = s * E_PER_TILE

        pltpu.async_copy(edges_hbm.at[0, pl.ds(tile_e0, CHUNK)], srcb, sem_s)

        def chunk(i, carry):
            e0 = tile_e0 + i * CHUNK
            pltpu.async_copy(edges_hbm.at[1, pl.ds(e0, CHUNK)], dstb, sem_d)
            pltpu.make_async_copy(edges_hbm.at[0, pl.ds(e0, CHUNK)], srcb,
                                  sem_s).wait()
            pltpu.sync_copy(xsh.at[srcb], valb)

            @pl.when(i < N_CHUNKS - 1)
            def _():
                pltpu.async_copy(edges_hbm.at[0, pl.ds(e0 + CHUNK, CHUNK)],
                                 srcb, sem_s)

            pltpu.make_async_copy(edges_hbm.at[1, pl.ds(e0, CHUNK)], dstb,
                                  sem_d).wait()
            pltpu.sync_copy(valb, aggsh.at[dstb], add=True)
            return carry

        lax.fori_loop(0, N_CHUNKS, chunk, 0)
        plsc.subcore_barrier()
        for k in range(4):
            p0 = n0 + k * HALF
            slot, sem = slots[k % 2], sems[k % 2]
            if k >= 2:
                pltpu.make_async_copy(
                    slot,
                    out_hbm.at[pl.ds(c * N + n0 + (k - 2) * HALF, HALF)],
                    sem).wait()
            pltpu.async_copy(aggsh.at[pl.ds(p0, HALF)], slot, sem).wait()
            pltpu.async_copy(slot, out_hbm.at[pl.ds(c * N + p0, HALF)], sem)
        for k in range(2, 4):
            pltpu.make_async_copy(
                slots[k % 2],
                out_hbm.at[pl.ds(c * N + n0 + k * HALF, HALF)],
                sems[k % 2]).wait()

    f = pl.kernel(
        body,
        out_type=jax.ShapeDtypeStruct((2 * N,), jnp.float32),
        mesh=mesh,
        scratch_types=[
            pltpu.VMEM_SHARED((N,), jnp.float32),
            pltpu.VMEM_SHARED((N,), jnp.float32),
            pltpu.VMEM((CHUNK,), jnp.int32),
            pltpu.VMEM((CHUNK,), jnp.int32),
            pltpu.VMEM((CHUNK,), jnp.float32),
            pltpu.SemaphoreType.DMA,
            pltpu.SemaphoreType.DMA,
        ],
    )
    return f(x1, x2, edges_r)


def _tc_head(agg, others, gcn_w, gcn_b, fc1_w, fc1_b, fc2_w, fc2_b,
             mlp1_w, mlp1_b, mlp2_w, mlp2_b, interpret=False):
    def body(agg_ref, oth_ref, gw_ref, gb_ref, w1_ref, b1_ref, w2_ref, b2_ref,
             m1w_ref, m1b_ref, m2w_ref, m2b_ref, out_ref):
        gw = gw_ref[...]
        gb = gb_ref[...]

        def enc(a):
            h = jnp.maximum(a * gw + gb, 0.0)
            h = lax.dot_general(h, w1_ref[...], (((1,), (1,)), ((), ())),
                                preferred_element_type=jnp.float32)
            h = jnp.maximum(h + b1_ref[...], 0.0)
            o = lax.dot_general(h, w2_ref[...], (((1,), (1,)), ((), ())),
                                preferred_element_type=jnp.float32)
            return o + b2_ref[...]

        o1 = enc(agg_ref[0])
        o2 = enc(agg_ref[1])
        p1 = o1 - jnp.mean(o1, axis=1, keepdims=True)
        p2 = o2 - jnp.mean(o2, axis=1, keepdims=True)
        n1 = jnp.sum(p1 * p1, axis=1, keepdims=True)
        n2 = jnp.sum(p2 * p2, axis=1, keepdims=True)
        p12 = jnp.sum(p1 * p2, axis=1, keepdims=True)
        r = p12 / jnp.sqrt(n1 * n2)
        r2 = r * r
        cat = jnp.concatenate([r2, oth_ref[...]], axis=1)
        z = lax.dot_general(cat, m1w_ref[...], (((1,), (1,)), ((), ())),
                            preferred_element_type=jnp.float32)
        z = jnp.maximum(z + m1b_ref[...], 0.0)
        out = lax.dot_general(z, m2w_ref[...], (((1,), (1,)), ((), ())),
                              preferred_element_type=jnp.float32)
        out_ref[...] = out + m2b_ref[...]

    return pl.pallas_call(
        body,
        out_shape=jax.ShapeDtypeStruct((B, 2), jnp.float32),
        interpret=interpret,
    )(agg, others, gcn_w, gcn_b, fc1_w, fc1_b, fc2_w, fc2_b,
      mlp1_w, mlp1_b, mlp2_w, mlp2_b)


def kernel(input1, input2, edges, input_others, gcn_w, gcn_b,
           fc1_w, fc1_b, fc2_w, fc2_b, mlp1_w, mlp1_b, mlp2_w, mlp2_b):
    x1 = input1.reshape(-1)
    x2 = input2.reshape(-1)
    agg = _sc_segment_sum(x1, x2, edges)
    return _tc_head(agg.reshape(2, B, G), input_others,
                    gcn_w, gcn_b.reshape(1, 1),
                    fc1_w, fc1_b.reshape(1, -1),
                    fc2_w, fc2_b.reshape(1, -1),
                    mlp1_w, mlp1_b.reshape(1, -1),
                    mlp2_w, mlp2_b.reshape(1, -1))
